# baseline (speedup 1.0000x reference)
import functools

import jax
import jax.numpy as jnp
from jax.experimental import pallas as pl
from jax.experimental.pallas import tpu as pltpu
from jax.experimental.pallas import tpu_sc as plsc

DIM = 768
NUM_EXPERTS = 64
CAPACITY_FACTOR = 1.0
EPSILON = 1e-06

BM = 4096
NC = 2
NS = 16
CHUNK = 128
CW = CHUNK * NUM_EXPERTS


def _stage_a(x_ref, w_ref, b_ref, score_ref, amax_ref, denom_ref):
    j = pl.program_id(0)
    xb = x_ref[0]
    logits = jax.lax.dot_general(
        w_ref[...], xb,
        (((1,), (1,)), ((), ())),
        preferred_element_type=jnp.float32,
    ) + b_ref[...]
    m = jnp.max(logits, axis=0, keepdims=True)
    idx = jax.lax.broadcasted_iota(jnp.int32, logits.shape, 0)
    a = jnp.min(jnp.where(logits == m, idx, NUM_EXPERTS), axis=0)
    s = jnp.sum(jnp.exp(logits - m), axis=0)
    score = 1.0 / s
    score_ref[0, 0, :] = score
    amax_ref[0, 0, :] = a
    onehot = (idx == a[None, :]).astype(jnp.float32)
    contrib = jax.lax.dot_general(
        onehot * score[None, :], jnp.ones((BM, 1), jnp.float32),
        (((1,), (0,)), ((), ())),
        preferred_element_type=jnp.float32,
    )

    @pl.when(j == 0)
    def _():
        denom_ref[...] = jnp.zeros_like(denom_ref)

    denom_ref[...] += contrib


def _make_sc_expand(rows):
    rpt = rows // (NC * NS)
    nchunk = rpt // CHUNK
    cap = float(rows * CAPACITY_FACTOR)

    def body(score_hbm, amax_hbm, denom_hbm, out_hbm,
             score_v, amax_v, denom_v, inv_v, buf, sem0, sem1):
        wid = jax.lax.axis_index("s") * NC + jax.lax.axis_index("c")
        base = wid * rpt
        pltpu.sync_copy(score_hbm.at[pl.ds(base, rpt)], score_v)
        pltpu.sync_copy(amax_hbm.at[pl.ds(base, rpt)], amax_v)
        pltpu.sync_copy(denom_hbm, denom_v)
        for t in range(NUM_EXPERTS // 16):
            d = denom_v[pl.ds(t * 16, 16)]
            inv_v[pl.ds(t * 16, 16)] = cap / (d + EPSILON)

        zeros16 = jnp.zeros((16,), jnp.float32)

        def zbody(i, carry):
            b = i * 128
            for u in range(8):
                buf[pl.ds(b + u * 16, 16)] = zeros16
            return carry

        jax.lax.fori_loop(0, 2 * CW // 128, zbody, 0)

        iota64 = jax.lax.iota(jnp.int32, 16) * NUM_EXPERTS

        def group_offs(k, g):
            rb = k * CHUNK + g * 16
            a16 = amax_v[pl.ds(rb, 16)]
            offs = (k % 2) * CW + g * 16 * NUM_EXPERTS + iota64 + a16
            return offs, rb, a16

        copies = [None, None]
        for k in range(nchunk):
            c = k % 2
            if k >= 2:
                copies[c].wait()
                for g in range(CHUNK // 16):
                    offs, _, _ = group_offs(k - 2, g)
                    plsc.store_scatter(buf, [offs], zeros16)
            for g in range(CHUNK // 16):
                offs, rb, a16 = group_offs(k, g)
                val = score_v[pl.ds(rb, 16)] * plsc.load_gather(inv_v, [a16])
                plsc.store_scatter(buf, [offs], val)
            sem = sem0 if c == 0 else sem1
            copies[c] = pltpu.async_copy(
                buf.at[pl.ds(c * CW, CW)],
                out_hbm.at[pl.ds((base + k * CHUNK) * NUM_EXPERTS, CW)],
                sem,
            )
        copies[nchunk % 2].wait()
        copies[(nchunk + 1) % 2].wait()

    return functools.partial(
        pl.kernel,
        mesh=plsc.VectorSubcoreMesh(core_axis_name="c", subcore_axis_name="s"),
        compiler_params=pltpu.CompilerParams(needs_layout_passes=False),
        out_type=jax.ShapeDtypeStruct((rows * NUM_EXPERTS,), jnp.float32),
        scratch_types=[
            pltpu.VMEM((rpt,), jnp.float32),
            pltpu.VMEM((rpt,), jnp.int32),
            pltpu.VMEM((NUM_EXPERTS,), jnp.float32),
            pltpu.VMEM((NUM_EXPERTS,), jnp.float32),
            pltpu.VMEM((2 * CW,), jnp.float32),
            pltpu.SemaphoreType.DMA,
            pltpu.SemaphoreType.DMA,
        ],
    )(body)


def kernel(x, W, b):
    batch, N, dim = x.shape
    rows = batch * N
    nb = rows // BM
    per_batch = N // BM
    b2 = b.reshape(NUM_EXPERTS, 1)

    score, amax, denom = pl.pallas_call(
        _stage_a,
        grid=(nb,),
        in_specs=[
            pl.BlockSpec((1, BM, dim), lambda j: (j // per_batch, j % per_batch, 0)),
            pl.BlockSpec((NUM_EXPERTS, dim), lambda j: (0, 0)),
            pl.BlockSpec((NUM_EXPERTS, 1), lambda j: (0, 0)),
        ],
        out_specs=[
            pl.BlockSpec((1, 1, BM), lambda j: (j, 0, 0)),
            pl.BlockSpec((1, 1, BM), lambda j: (j, 0, 0)),
            pl.BlockSpec((NUM_EXPERTS, 1), lambda j: (0, 0)),
        ],
        out_shape=[
            jax.ShapeDtypeStruct((nb, 1, BM), jnp.float32),
            jax.ShapeDtypeStruct((nb, 1, BM), jnp.int32),
            jax.ShapeDtypeStruct((NUM_EXPERTS, 1), jnp.float32),
        ],
    )(x, W, b2)

    expand = _make_sc_expand(rows)
    out_flat = expand(
        score.reshape(rows), amax.reshape(rows), denom.reshape(NUM_EXPERTS)
    )
    return out_flat.reshape(batch, N, NUM_EXPERTS)


# speedup vs baseline: 1.1390x; 1.1390x over previous
import functools

import jax
import jax.numpy as jnp
from jax.experimental import pallas as pl
from jax.experimental.pallas import tpu as pltpu
from jax.experimental.pallas import tpu_sc as plsc

DIM = 768
NUM_EXPERTS = 64
CAPACITY_FACTOR = 1.0
EPSILON = 1e-06

BM = 4096
NC = 2
NS = 16
CHUNK = 128
CW = CHUNK * NUM_EXPERTS


def _stage_a(x_ref, w_ref, b_ref, score_ref, amax_ref, denom_ref):
    j = pl.program_id(0)
    xb = x_ref[0]
    logits = jax.lax.dot_general(
        w_ref[...], xb,
        (((1,), (1,)), ((), ())),
        preferred_element_type=jnp.float32,
    ) + b_ref[...]
    m = jnp.max(logits, axis=0, keepdims=True)
    idx = jax.lax.broadcasted_iota(jnp.int32, logits.shape, 0)
    a = jnp.min(jnp.where(logits == m, idx, NUM_EXPERTS), axis=0)
    s = jnp.sum(jnp.exp(logits - m), axis=0)
    score = 1.0 / s
    score_ref[0, 0, :] = score
    amax_ref[0, 0, :] = a
    onehot = (idx == a[None, :]).astype(jnp.float32)
    contrib = jax.lax.dot_general(
        onehot * score[None, :], jnp.ones((BM, 1), jnp.float32),
        (((1,), (0,)), ((), ())),
        preferred_element_type=jnp.float32,
    )

    @pl.when(j == 0)
    def _():
        denom_ref[...] = jnp.zeros_like(denom_ref)

    denom_ref[...] += contrib


def _make_sc_expand(rows):
    rpt = rows // (NC * NS)
    nchunk = rpt // CHUNK
    cap = float(rows * CAPACITY_FACTOR)

    def body(score_hbm, amax_hbm, denom_hbm, out_hbm,
             score_v, amax_v, denom_v, inv_v, buf, sem0, sem1):
        wid = jax.lax.axis_index("s") * NC + jax.lax.axis_index("c")
        base = wid * rpt
        pltpu.sync_copy(score_hbm.at[pl.ds(base, rpt)], score_v)
        pltpu.sync_copy(amax_hbm.at[pl.ds(base, rpt)], amax_v)
        pltpu.sync_copy(denom_hbm, denom_v)
        for t in range(NUM_EXPERTS // 16):
            d = denom_v[pl.ds(t * 16, 16)]
            inv_v[pl.ds(t * 16, 16)] = cap / (d + EPSILON)

        zeros16 = jnp.zeros((16,), jnp.float32)
        iota16 = jax.lax.iota(jnp.int32, 16)

        def zbody(r, carry):
            for c in range(2):
                for u in range(NUM_EXPERTS // 16):
                    buf[c, r, pl.ds(u * 16, 16)] = zeros16
            return carry

        jax.lax.fori_loop(0, CHUNK, zbody, 0)

        def group_idx(k, g):
            rb = k * CHUNK + g * 16
            a16 = amax_v[pl.ds(rb, 16)]
            c16 = jnp.full((16,), k % 2, jnp.int32)
            r16 = g * 16 + iota16
            return [c16, r16, a16], rb, a16

        copies = [None, None]
        for k in range(nchunk):
            c = k % 2
            if k >= 2:
                copies[c].wait()
                for g in range(CHUNK // 16):
                    idx3, _, _ = group_idx(k - 2, g)
                    plsc.store_scatter(buf, idx3, zeros16)
            for g in range(CHUNK // 16):
                idx3, rb, a16 = group_idx(k, g)
                val = score_v[pl.ds(rb, 16)] * plsc.load_gather(inv_v, [a16])
                plsc.store_scatter(buf, idx3, val)
            sem = sem0 if c == 0 else sem1
            copies[c] = pltpu.async_copy(
                buf.at[c],
                out_hbm.at[pl.ds(base + k * CHUNK, CHUNK)],
                sem,
            )
        copies[nchunk % 2].wait()
        copies[(nchunk + 1) % 2].wait()

    return functools.partial(
        pl.kernel,
        mesh=plsc.VectorSubcoreMesh(core_axis_name="c", subcore_axis_name="s"),
        compiler_params=pltpu.CompilerParams(needs_layout_passes=False),
        out_type=jax.ShapeDtypeStruct((rows, NUM_EXPERTS), jnp.float32),
        scratch_types=[
            pltpu.VMEM((rpt,), jnp.float32),
            pltpu.VMEM((rpt,), jnp.int32),
            pltpu.VMEM((NUM_EXPERTS,), jnp.float32),
            pltpu.VMEM((NUM_EXPERTS,), jnp.float32),
            pltpu.VMEM((2, CHUNK, NUM_EXPERTS), jnp.float32),
            pltpu.SemaphoreType.DMA,
            pltpu.SemaphoreType.DMA,
        ],
    )(body)


def kernel(x, W, b):
    batch, N, dim = x.shape
    rows = batch * N
    nb = rows // BM
    per_batch = N // BM
    b2 = b.reshape(NUM_EXPERTS, 1)

    score, amax, denom = pl.pallas_call(
        _stage_a,
        grid=(nb,),
        in_specs=[
            pl.BlockSpec((1, BM, dim), lambda j: (j // per_batch, j % per_batch, 0)),
            pl.BlockSpec((NUM_EXPERTS, dim), lambda j: (0, 0)),
            pl.BlockSpec((NUM_EXPERTS, 1), lambda j: (0, 0)),
        ],
        out_specs=[
            pl.BlockSpec((1, 1, BM), lambda j: (j, 0, 0)),
            pl.BlockSpec((1, 1, BM), lambda j: (j, 0, 0)),
            pl.BlockSpec((NUM_EXPERTS, 1), lambda j: (0, 0)),
        ],
        out_shape=[
            jax.ShapeDtypeStruct((nb, 1, BM), jnp.float32),
            jax.ShapeDtypeStruct((nb, 1, BM), jnp.int32),
            jax.ShapeDtypeStruct((NUM_EXPERTS, 1), jnp.float32),
        ],
    )(x, W, b2)

    expand = _make_sc_expand(rows)
    out2d = expand(
        score.reshape(rows), amax.reshape(rows), denom.reshape(NUM_EXPERTS)
    )
    return out2d.reshape(batch, N, NUM_EXPERTS)


# speedup vs baseline: 1.3839x; 1.2150x over previous
import functools

import jax
import jax.numpy as jnp
from jax.experimental import pallas as pl
from jax.experimental.pallas import tpu as pltpu
from jax.experimental.pallas import tpu_sc as plsc

DIM = 768
NUM_EXPERTS = 64
CAPACITY_FACTOR = 1.0
EPSILON = 1e-06

BM = 4096
NC = 2
NS = 16
CHUNK = 128
CW = CHUNK * NUM_EXPERTS


def _stage_a(x_ref, w_ref, b_ref, score_ref, amax_ref, denom_ref):
    j = pl.program_id(0)
    xb = x_ref[0]
    logits = jax.lax.dot_general(
        w_ref[...], xb,
        (((1,), (1,)), ((), ())),
        preferred_element_type=jnp.float32,
    ) + b_ref[...]
    m = jnp.max(logits, axis=0, keepdims=True)
    idx = jax.lax.broadcasted_iota(jnp.int32, logits.shape, 0)
    a = jnp.min(jnp.where(logits == m, idx, NUM_EXPERTS), axis=0)
    s = jnp.sum(jnp.exp(logits - m), axis=0)
    score = 1.0 / s
    score_ref[0, 0, :] = score
    amax_ref[0, 0, :] = a
    onehot = (idx == a[None, :]).astype(jnp.float32)
    contrib = jax.lax.dot_general(
        onehot * score[None, :], jnp.ones((BM, 1), jnp.float32),
        (((1,), (0,)), ((), ())),
        preferred_element_type=jnp.float32,
    )

    @pl.when(j == 0)
    def _():
        denom_ref[...] = jnp.zeros_like(denom_ref)

    denom_ref[...] += contrib


def _make_sc_expand(batch, n_tok):
    rows = batch * n_tok
    rpt = rows // (NC * NS)
    tiles_per_b = n_tok // rpt
    nchunk = rpt // CHUNK
    cap = float(rows * CAPACITY_FACTOR)

    def body(score_hbm, amax_hbm, denom_hbm, out_hbm,
             score_v, amax_v, denom_v, inv_v, buf, sem0, sem1):
        wid = jax.lax.axis_index("s") * NC + jax.lax.axis_index("c")
        base = wid * rpt
        bidx = wid // tiles_per_b
        n0 = (wid % tiles_per_b) * rpt
        pltpu.sync_copy(score_hbm.at[pl.ds(base, rpt)], score_v)
        pltpu.sync_copy(amax_hbm.at[pl.ds(base, rpt)], amax_v)
        pltpu.sync_copy(denom_hbm, denom_v)
        for t in range(NUM_EXPERTS // 16):
            d = denom_v[pl.ds(t * 16, 16)]
            inv_v[pl.ds(t * 16, 16)] = cap / (d + EPSILON)

        zeros16 = jnp.zeros((16,), jnp.float32)
        iota16 = jax.lax.iota(jnp.int32, 16)

        def zbody(e, carry):
            for c in range(2):
                for u in range(CHUNK // 16):
                    buf[c, e, pl.ds(u * 16, 16)] = zeros16
            return carry

        jax.lax.fori_loop(0, NUM_EXPERTS, zbody, 0)

        def group_idx(k, g):
            rb = k * CHUNK + g * 16
            a16 = amax_v[pl.ds(rb, 16)]
            c16 = jnp.full((16,), k % 2, jnp.int32)
            r16 = g * 16 + iota16
            return [c16, a16, r16], rb, a16

        copies = [None, None]
        for k in range(nchunk):
            c = k % 2
            if k >= 2:
                copies[c].wait()
                for g in range(CHUNK // 16):
                    idx3, _, _ = group_idx(k - 2, g)
                    plsc.store_scatter(buf, idx3, zeros16)
            for g in range(CHUNK // 16):
                idx3, rb, a16 = group_idx(k, g)
                val = score_v[pl.ds(rb, 16)] * plsc.load_gather(inv_v, [a16])
                plsc.store_scatter(buf, idx3, val)
            sem = sem0 if c == 0 else sem1
            copies[c] = pltpu.async_copy(
                buf.at[c],
                out_hbm.at[bidx, :, pl.ds(n0 + k * CHUNK, CHUNK)],
                sem,
            )
        copies[nchunk % 2].wait()
        copies[(nchunk + 1) % 2].wait()

    return functools.partial(
        pl.kernel,
        mesh=plsc.VectorSubcoreMesh(core_axis_name="c", subcore_axis_name="s"),
        compiler_params=pltpu.CompilerParams(needs_layout_passes=False),
        out_type=jax.ShapeDtypeStruct((batch, NUM_EXPERTS, n_tok), jnp.float32),
        scratch_types=[
            pltpu.VMEM((rpt,), jnp.float32),
            pltpu.VMEM((rpt,), jnp.int32),
            pltpu.VMEM((NUM_EXPERTS,), jnp.float32),
            pltpu.VMEM((NUM_EXPERTS,), jnp.float32),
            pltpu.VMEM((2, NUM_EXPERTS, CHUNK), jnp.float32),
            pltpu.SemaphoreType.DMA,
            pltpu.SemaphoreType.DMA,
        ],
    )(body)


def kernel(x, W, b):
    batch, N, dim = x.shape
    rows = batch * N
    nb = rows // BM
    per_batch = N // BM
    b2 = b.reshape(NUM_EXPERTS, 1)

    score, amax, denom = pl.pallas_call(
        _stage_a,
        grid=(nb,),
        in_specs=[
            pl.BlockSpec((1, BM, dim), lambda j: (j // per_batch, j % per_batch, 0)),
            pl.BlockSpec((NUM_EXPERTS, dim), lambda j: (0, 0)),
            pl.BlockSpec((NUM_EXPERTS, 1), lambda j: (0, 0)),
        ],
        out_specs=[
            pl.BlockSpec((1, 1, BM), lambda j: (j, 0, 0)),
            pl.BlockSpec((1, 1, BM), lambda j: (j, 0, 0)),
            pl.BlockSpec((NUM_EXPERTS, 1), lambda j: (0, 0)),
        ],
        out_shape=[
            jax.ShapeDtypeStruct((nb, 1, BM), jnp.float32),
            jax.ShapeDtypeStruct((nb, 1, BM), jnp.int32),
            jax.ShapeDtypeStruct((NUM_EXPERTS, 1), jnp.float32),
        ],
    )(x, W, b2)

    expand = _make_sc_expand(batch, N)
    out_t = expand(
        score.reshape(rows), amax.reshape(rows), denom.reshape(NUM_EXPERTS)
    )
    return jnp.transpose(out_t, (0, 2, 1))
